# baseline (speedup 1.0000x reference)
import jax
import jax.numpy as jnp
from jax.experimental import pallas as pl
from jax.experimental.pallas import tpu as pltpu


_N_CHUNKS = 16


def _identity_copy_kernel(src_ref, dst_ref, buf_ref, in_sems, out_sems):
    for i in range(_N_CHUNKS):
        pltpu.make_async_copy(src_ref.at[i], buf_ref.at[i], in_sems.at[i]).start()
    for i in range(_N_CHUNKS):
        pltpu.make_async_copy(src_ref.at[i], buf_ref.at[i], in_sems.at[i]).wait()
        pltpu.make_async_copy(buf_ref.at[i], dst_ref.at[i], out_sems.at[i]).start()
    for i in range(_N_CHUNKS):
        pltpu.make_async_copy(buf_ref.at[i], dst_ref.at[i], out_sems.at[i]).wait()


def kernel(z, embedding):
    del embedding
    rows = z.shape[0] * z.shape[1]
    zc = z.reshape(_N_CHUNKS, rows // _N_CHUNKS, z.shape[2])
    out = pl.pallas_call(
        _identity_copy_kernel,
        out_shape=jax.ShapeDtypeStruct(zc.shape, zc.dtype),
        in_specs=[pl.BlockSpec(memory_space=pl.ANY)],
        out_specs=pl.BlockSpec(memory_space=pl.ANY),
        scratch_shapes=[
            pltpu.VMEM(zc.shape, zc.dtype),
            pltpu.SemaphoreType.DMA((_N_CHUNKS,)),
            pltpu.SemaphoreType.DMA((_N_CHUNKS,)),
        ],
    )(zc)
    return out.reshape(z.shape)


# speedup vs baseline: 1.0218x; 1.0218x over previous
import jax
import jax.numpy as jnp
from jax.experimental import pallas as pl
from jax.experimental.pallas import tpu as pltpu


_N_CHUNKS = 2


def _identity_copy_kernel(src_ref, dst_ref, buf_ref, in_sems, out_sems):
    for i in range(_N_CHUNKS):
        pltpu.make_async_copy(src_ref.at[i], buf_ref.at[i], in_sems.at[i]).start()
    for i in range(_N_CHUNKS):
        pltpu.make_async_copy(src_ref.at[i], buf_ref.at[i], in_sems.at[i]).wait()
        pltpu.make_async_copy(buf_ref.at[i], dst_ref.at[i], out_sems.at[i]).start()
    for i in range(_N_CHUNKS):
        pltpu.make_async_copy(buf_ref.at[i], dst_ref.at[i], out_sems.at[i]).wait()


def kernel(z, embedding):
    del embedding
    rows = z.shape[0] * z.shape[1]
    zc = z.reshape(_N_CHUNKS, rows // _N_CHUNKS, z.shape[2])
    out = pl.pallas_call(
        _identity_copy_kernel,
        out_shape=jax.ShapeDtypeStruct(zc.shape, zc.dtype),
        in_specs=[pl.BlockSpec(memory_space=pl.ANY)],
        out_specs=pl.BlockSpec(memory_space=pl.ANY),
        scratch_shapes=[
            pltpu.VMEM(zc.shape, zc.dtype),
            pltpu.SemaphoreType.DMA((_N_CHUNKS,)),
            pltpu.SemaphoreType.DMA((_N_CHUNKS,)),
        ],
    )(zc)
    return out.reshape(z.shape)


# speedup vs baseline: 1.0257x; 1.0038x over previous
import jax
import jax.numpy as jnp
from jax.experimental import pallas as pl
from jax.experimental.pallas import tpu as pltpu


_N_CHUNKS = 8


def _identity_copy_kernel(src_ref, dst_ref, buf_ref, in_sems, out_sems):
    for i in range(_N_CHUNKS):
        pltpu.make_async_copy(src_ref.at[i], buf_ref.at[i], in_sems.at[i]).start()
    for i in range(_N_CHUNKS):
        pltpu.make_async_copy(src_ref.at[i], buf_ref.at[i], in_sems.at[i]).wait()
        pltpu.make_async_copy(buf_ref.at[i], dst_ref.at[i], out_sems.at[i]).start()
    for i in range(_N_CHUNKS):
        pltpu.make_async_copy(buf_ref.at[i], dst_ref.at[i], out_sems.at[i]).wait()


def kernel(z, embedding):
    del embedding
    rows = z.shape[0] * z.shape[1]
    zc = z.reshape(_N_CHUNKS, rows // _N_CHUNKS, z.shape[2])
    out = pl.pallas_call(
        _identity_copy_kernel,
        out_shape=jax.ShapeDtypeStruct(zc.shape, zc.dtype),
        in_specs=[pl.BlockSpec(memory_space=pl.ANY)],
        out_specs=pl.BlockSpec(memory_space=pl.ANY),
        scratch_shapes=[
            pltpu.VMEM(zc.shape, zc.dtype),
            pltpu.SemaphoreType.DMA((_N_CHUNKS,)),
            pltpu.SemaphoreType.DMA((_N_CHUNKS,)),
        ],
        compiler_params=pltpu.CompilerParams(skip_device_barrier=True),
    )(zc)
    return out.reshape(z.shape)


# speedup vs baseline: 1.0328x; 1.0069x over previous
import jax
from jax.experimental import pallas as pl
from jax.experimental.pallas import tpu as pltpu


_N_CHUNKS = 8


def _identity_copy_kernel(src_ref, dst_ref, buf_ref, in_sems, out_sems):
    for i in range(_N_CHUNKS):
        pltpu.make_async_copy(src_ref.at[i], buf_ref.at[i], in_sems.at[i]).start()
    for i in range(_N_CHUNKS):
        pltpu.make_async_copy(src_ref.at[i], buf_ref.at[i], in_sems.at[i]).wait()
        pltpu.make_async_copy(buf_ref.at[i], dst_ref.at[i], out_sems.at[i]).start()
    for i in range(_N_CHUNKS):
        pltpu.make_async_copy(buf_ref.at[i], dst_ref.at[i], out_sems.at[i]).wait()


def kernel(z, embedding):
    del embedding
    rows = z.shape[0] * z.shape[1]
    zc = z.reshape(_N_CHUNKS, rows // _N_CHUNKS, z.shape[2])
    out = pl.pallas_call(
        _identity_copy_kernel,
        out_shape=jax.ShapeDtypeStruct(zc.shape, zc.dtype),
        in_specs=[pl.BlockSpec(memory_space=pl.ANY)],
        out_specs=pl.BlockSpec(memory_space=pl.ANY),
        scratch_shapes=[
            pltpu.VMEM(zc.shape, zc.dtype),
            pltpu.SemaphoreType.DMA((_N_CHUNKS,)),
            pltpu.SemaphoreType.DMA((_N_CHUNKS,)),
        ],
    )(zc)
    return out.reshape(z.shape)
